# baseline (device time: 480409 ns/iter reference)
import jax
import jax.numpy as jnp
from jax import lax
from jax.experimental import pallas as pl
from jax.experimental.pallas import tpu as pltpu

N_EXPERTS = 8
E_LOCAL = 4
C = 576
F_TILES = 8


def _moe_body(xp_ref, w1_ref, w2_ref, out_ref,
              recv_x, xl, y_stage, y_send, acc_l, acc_r,
              d_send_sems, d_recv_sems, c_send_sems, c_recv_sems,
              lx_sem, lo_sem):
    j = pl.program_id(0)
    k = pl.program_id(1)
    my_x = lax.axis_index("x")
    my_y = lax.axis_index("y")
    my_z = lax.axis_index("z")
    peer = (1 - my_x, my_y, my_z)

    def dispatch_rdma(jj, sx):
        return pltpu.make_async_remote_copy(
            src_ref=xp_ref.at[E_LOCAL * (1 - sx) + jj],
            dst_ref=recv_x.at[jj],
            send_sem=d_send_sems.at[jj],
            recv_sem=d_recv_sems.at[jj],
            device_id=peer,
            device_id_type=pl.DeviceIdType.MESH,
        )

    def combine_rdma(jj, sx):
        return pltpu.make_async_remote_copy(
            src_ref=y_send.at[jj % 2],
            dst_ref=out_ref.at[E_LOCAL * sx + jj],
            send_sem=c_send_sems.at[jj],
            recv_sem=c_recv_sems.at[jj],
            device_id=peer,
            device_id_type=pl.DeviceIdType.MESH,
        )

    def for_my_x(fn):
        for sx in (0, 1):
            @pl.when(my_x == sx)
            def _(sx=sx):
                fn(sx)

    @pl.when(jnp.logical_and(j == 0, k == 0))
    def _():
        barrier_sem = pltpu.get_barrier_semaphore()
        pl.semaphore_signal(
            barrier_sem, inc=1, device_id=peer,
            device_id_type=pl.DeviceIdType.MESH,
        )
        pl.semaphore_wait(barrier_sem, 1)

        def start_dispatch(sx):
            for jj in range(E_LOCAL):
                dispatch_rdma(jj, sx).start()
        for_my_x(start_dispatch)

    @pl.when(k == 0)
    def _():
        for jj in range(E_LOCAL):
            @pl.when(j == jj)
            def _(jj=jj):
                def load_local(sx):
                    cp = pltpu.make_async_copy(
                        xp_ref.at[E_LOCAL * sx + jj], xl, lx_sem)
                    cp.start()
                    cp.wait()
                for_my_x(load_local)
                dispatch_rdma(jj, 0).wait_recv()

    w1t = w1_ref[0].astype(jnp.bfloat16)
    w2t = w2_ref[0].astype(jnp.bfloat16)
    xlv = xl[...]
    xr = recv_x[pl.ds(j, 1)].reshape(xlv.shape)

    hl = jnp.maximum(jnp.dot(xlv, w1t, preferred_element_type=jnp.float32), 0.0)
    pl_part = jnp.dot(hl.astype(jnp.bfloat16), w2t,
                      preferred_element_type=jnp.float32)
    hr = jnp.maximum(jnp.dot(xr, w1t, preferred_element_type=jnp.float32), 0.0)
    pr_part = jnp.dot(hr.astype(jnp.bfloat16), w2t,
                      preferred_element_type=jnp.float32)

    @pl.when(k == 0)
    def _():
        acc_l[...] = pl_part
        acc_r[...] = pr_part

    @pl.when(k > 0)
    def _():
        acc_l[...] += pl_part
        acc_r[...] += pr_part

    @pl.when(k == F_TILES - 1)
    def _():
        for jj in range(E_LOCAL):
            @pl.when(j == jj)
            def _(jj=jj):
                y_stage[...] = acc_l[...].astype(jnp.bfloat16)

                def store_local(sx):
                    cp = pltpu.make_async_copy(
                        y_stage, out_ref.at[E_LOCAL * sx + jj], lo_sem)
                    cp.start()
                    cp.wait()
                for_my_x(store_local)

                if jj >= 2:
                    for_my_x(lambda sx, jj=jj: combine_rdma(jj - 2, sx).wait_send())
                y_send[jj % 2] = acc_r[...].astype(jnp.bfloat16)
                for_my_x(lambda sx, jj=jj: combine_rdma(jj, sx).start())

    @pl.when(jnp.logical_and(j == E_LOCAL - 1, k == F_TILES - 1))
    def _():
        def drain(sx):
            for jj in range(E_LOCAL):
                dispatch_rdma(jj, sx).wait_send()
            for jj in range(E_LOCAL - 2, E_LOCAL):
                combine_rdma(jj, sx).wait_send()
            for jj in range(E_LOCAL):
                pltpu.make_async_remote_copy(
                    src_ref=y_send.at[jj % 2],
                    dst_ref=out_ref.at[E_LOCAL * (1 - sx) + jj],
                    send_sem=c_send_sems.at[jj],
                    recv_sem=c_recv_sems.at[jj],
                    device_id=peer,
                    device_id_type=pl.DeviceIdType.MESH,
                ).wait_recv()
        for_my_x(drain)


def _pack_body(sbt_ref, x_ref, xp_ref):
    e = pl.program_id(0)
    sbt = sbt_ref[...].reshape(1, -1)
    rows = jax.lax.broadcasted_iota(jnp.int32, (C, sbt.shape[1]), 0)
    p_t = (rows + e * C == sbt).astype(jnp.bfloat16)
    xp_ref[0] = jnp.dot(p_t, x_ref[...],
                        preferred_element_type=jnp.float32).astype(jnp.bfloat16)


def _unpack_body(sbt_ref, y_ref, out_ref):
    i = pl.program_id(0)
    s = sbt_ref[pl.ds(i, 1)].reshape(-1, 1)
    cols = jax.lax.broadcasted_iota(
        jnp.int32, (s.shape[0], N_EXPERTS * C), 1)
    p = (cols == s).astype(jnp.bfloat16)
    yf = y_ref[...].reshape(N_EXPERTS * C, -1)
    out_ref[...] = jnp.dot(p, yf, preferred_element_type=jnp.float32)


_T_TILES = 8


def kernel(x, assign, W1, W2):
    T, d = x.shape
    f = W1.shape[2]
    ft = f // F_TILES
    tt = T // _T_TILES
    x16 = x.astype(jnp.bfloat16)

    oh = (assign[:, None] == jnp.arange(N_EXPERTS, dtype=assign.dtype)[None, :])
    ohi = oh.astype(jnp.int32)
    rank = jnp.sum(ohi * (jnp.cumsum(ohi, axis=0) - 1), axis=1)
    slot_by_token = (assign * C + jnp.minimum(rank, C - 1)).astype(jnp.int32)

    xp = pl.pallas_call(
        _pack_body,
        grid=(N_EXPERTS,),
        in_specs=[
            pl.BlockSpec((_T_TILES, tt), lambda e: (0, 0)),
            pl.BlockSpec((T, d), lambda e: (0, 0)),
        ],
        out_specs=pl.BlockSpec((1, C, d), lambda e: (e, 0, 0)),
        out_shape=jax.ShapeDtypeStruct((N_EXPERTS, C, d), jnp.bfloat16),
        compiler_params=pltpu.CompilerParams(
            vmem_limit_bytes=60 * 1024 * 1024,
        ),
    )(slot_by_token.reshape(_T_TILES, tt), x16)

    out8 = pl.pallas_call(
        _moe_body,
        grid=(E_LOCAL, F_TILES),
        in_specs=[
            pl.BlockSpec(memory_space=pl.ANY),
            pl.BlockSpec((1, d, ft), lambda j, k: (j, 0, k)),
            pl.BlockSpec((1, ft, d), lambda j, k: (j, k, 0)),
        ],
        out_specs=pl.BlockSpec(memory_space=pl.ANY),
        out_shape=jax.ShapeDtypeStruct((N_EXPERTS, C, d), jnp.bfloat16),
        scratch_shapes=[
            pltpu.VMEM((E_LOCAL, C, d), jnp.bfloat16),
            pltpu.VMEM((C, d), jnp.bfloat16),
            pltpu.VMEM((C, d), jnp.bfloat16),
            pltpu.VMEM((2, C, d), jnp.bfloat16),
            pltpu.VMEM((C, d), jnp.float32),
            pltpu.VMEM((C, d), jnp.float32),
            pltpu.SemaphoreType.DMA((E_LOCAL,)),
            pltpu.SemaphoreType.DMA((E_LOCAL,)),
            pltpu.SemaphoreType.DMA((E_LOCAL,)),
            pltpu.SemaphoreType.DMA((E_LOCAL,)),
            pltpu.SemaphoreType.DMA,
            pltpu.SemaphoreType.DMA,
        ],
        compiler_params=pltpu.CompilerParams(
            collective_id=0,
            vmem_limit_bytes=63 * 1024 * 1024,
        ),
    )(xp, W1, W2)

    out = pl.pallas_call(
        _unpack_body,
        grid=(_T_TILES,),
        in_specs=[
            pl.BlockSpec((_T_TILES, tt), lambda i: (0, 0)),
            pl.BlockSpec((N_EXPERTS, C, d), lambda i: (0, 0, 0)),
        ],
        out_specs=pl.BlockSpec((tt, d), lambda i: (i, 0)),
        out_shape=jax.ShapeDtypeStruct((T, d), jnp.float32),
        compiler_params=pltpu.CompilerParams(
            vmem_limit_bytes=60 * 1024 * 1024,
        ),
    )(slot_by_token.reshape(_T_TILES, tt), out8)
    return out
